# Initial kernel scaffold; baseline (speedup 1.0000x reference)
import jax
import jax.numpy as jnp
from jax.experimental import pallas as pl


def kernel(edge_index, types_tokens, node_types, strings, numbers, readout_idx, str_table, Wx, Wh, bh, W_num, b_num, W_bn, b_bn, W1, b1, W2, b2, W_head, b_head):
    raise NotImplementedError("write your pallas kernel here")


# speedup vs baseline: 3.1156x; 3.1156x over previous
import functools

import jax
import jax.numpy as jnp
from jax import lax
from jax.experimental import pallas as pl
from jax.experimental.pallas import tpu as pltpu
from jax.experimental.pallas import tpu_sc as plsc

N = 10000
D = 128
NC = 2
NS = 16
NW = NC * NS
N_PAD = 10240
DUMMY = N
E_PAD = 327680
ROWS_PER_TILE = N_PAD // NS
CHUNKS_PER_TILE = ROWS_PER_TILE // 128

_f32 = jnp.float32
_i32 = jnp.int32


def _mesh():
    return plsc.VectorSubcoreMesh(core_axis_name="c", subcore_axis_name="s")


def _zero_rows(buf, nrows, ncol16):
    def row(r, _):
        for q in range(ncol16):
            buf[r, pl.ds(q * 16, 16)] = jnp.zeros((16,), _f32)
        return 0
    lax.fori_loop(0, nrows, row, 0)


def _sc_gather(table, idx3, k):
    nout = NW * k * 128

    @functools.partial(
        pl.kernel,
        out_type=jax.ShapeDtypeStruct((nout, 128), _f32),
        mesh=_mesh(),
        scratch_types=[
            pltpu.VMEM((k, 128), _i32),
            pltpu.VMEM((128, 128), _f32),
            pltpu.SemaphoreType.DMA,
        ],
    )
    def kfn(table_h, idx_h, out_h, idx_v, buf, sem):
        cid = lax.axis_index("c")
        sid = lax.axis_index("s")
        w = cid * NS + sid
        pltpu.sync_copy(idx_h.at[w], idx_v)
        for j in range(k):
            pltpu.async_copy(table_h.at[idx_v.at[j]], buf, sem).wait()
            pltpu.sync_copy(buf, out_h.at[pl.ds((w * k + j) * 128, 128)])

    return kfn(table, idx3)


def _sc_deg(dst16):
    @functools.partial(
        pl.kernel,
        out_type=jax.ShapeDtypeStruct((N_PAD, 128), _f32),
        mesh=_mesh(),
        scratch_types=[
            pltpu.VMEM_SHARED((N_PAD, 16), _f32),
            pltpu.VMEM((160, 128), _i32),
            pltpu.VMEM((128, 16), _f32),
            pltpu.VMEM((128, 16), _f32),
            pltpu.VMEM((128, 128), _f32),
            pltpu.SemaphoreType.DMA,
        ],
    )
    def kfn(dst_h, out_h, deg_sh, dst_v, ones_v, tmp16, invb, sem):
        cid = lax.axis_index("c")
        sid = lax.axis_index("s")

        @pl.when(cid == 0)
        def _():
            def initrow(r, _):
                ones_v[r, :] = jnp.ones((16,), _f32)
                tmp16[r, :] = jnp.zeros((16,), _f32)
                return 0
            lax.fori_loop(0, 128, initrow, 0)
            base = sid * ROWS_PER_TILE
            for c in range(CHUNKS_PER_TILE):
                pltpu.sync_copy(tmp16, deg_sh.at[pl.ds(base + c * 128, 128)])
            plsc.subcore_barrier()
            pltpu.sync_copy(dst_h.at[sid], dst_v)

            def body(j, _):
                pltpu.sync_copy(ones_v, deg_sh.at[dst_v.at[j]], add=True)
                return 0
            lax.fori_loop(0, 160, body, 0)
            plsc.subcore_barrier()
            for c in range(CHUNKS_PER_TILE):
                pltpu.sync_copy(deg_sh.at[pl.ds(base + c * 128, 128)], tmp16)

                def brow(r, _):
                    inv = 1.0 / (1.0 + tmp16[r, :])
                    for q in range(8):
                        invb[r, pl.ds(q * 16, 16)] = inv
                    return 0
                lax.fori_loop(0, 128, brow, 0)
                pltpu.sync_copy(invb, out_h.at[pl.ds(base + c * 128, 128)])

    return kfn(dst16)


def _sc_msg(h, src3, dst3):
    @functools.partial(
        pl.kernel,
        out_type=jax.ShapeDtypeStruct((NC, N_PAD, 128), _f32),
        mesh=_mesh(),
        scratch_types=[
            pltpu.VMEM_SHARED((N_PAD, 128), _f32),
            pltpu.VMEM((80, 128), _i32),
            pltpu.VMEM((80, 128), _i32),
            pltpu.VMEM((128, 128), _f32),
            pltpu.SemaphoreType.DMA,
        ],
    )
    def kfn(h_h, src_h, dst_h, out_h, acc_sh, src_v, dst_v, buf, sem):
        cid = lax.axis_index("c")
        sid = lax.axis_index("s")
        w = cid * NS + sid
        _zero_rows(buf, 128, 8)
        base = sid * ROWS_PER_TILE
        for c in range(CHUNKS_PER_TILE):
            pltpu.sync_copy(buf, acc_sh.at[pl.ds(base + c * 128, 128)])
        plsc.subcore_barrier()
        pltpu.sync_copy(src_h.at[w], src_v)
        pltpu.sync_copy(dst_h.at[w], dst_v)

        def body(j, _):
            pltpu.async_copy(h_h.at[src_v.at[j]], buf, sem).wait()
            pltpu.sync_copy(buf, acc_sh.at[dst_v.at[j]], add=True)
            return 0
        lax.fori_loop(0, 80, body, 0)
        plsc.subcore_barrier()
        for c in range(CHUNKS_PER_TILE):
            pltpu.sync_copy(acc_sh.at[pl.ds(base + c * 128, 128)],
                            out_h.at[cid, pl.ds(base + c * 128, 128)])

    return kfn(h, src3, dst3)


def _tc_rnn(tok, Wx, Wh, bh2):
    def body(tok_ref, wx_ref, wh_ref, b_ref, out_ref):
        wx = wx_ref[...]
        wh = wh_ref[...]
        b = b_ref[...]
        h = jnp.zeros((512, 128), _f32)
        for l in range(8):
            x = tok_ref[pl.ds(l * 512, 512), :]
            h = jnp.tanh(jnp.dot(x, wx, preferred_element_type=_f32)
                         + jnp.dot(h, wh, preferred_element_type=_f32) + b)
        out_ref[...] = h

    return pl.pallas_call(
        body, out_shape=jax.ShapeDtypeStruct((512, 128), _f32))(tok, Wx, Wh, bh2)


def _tc_bn(key_emb, str_e, nums2, Wb1, Wb2, wnum_row, cprime):
    def body(k_ref, s_ref, n_ref, w1_ref, w2_ref, wn_ref, c_ref, out_ref):
        acc = jnp.dot(k_ref[...], w1_ref[...], preferred_element_type=_f32)
        acc += jnp.dot(s_ref[...], w2_ref[...], preferred_element_type=_f32)
        acc += n_ref[...] * wn_ref[...]
        acc += c_ref[...]
        out_ref[...] = jnp.maximum(acc, 0.0)

    return pl.pallas_call(
        body, out_shape=jax.ShapeDtypeStruct((N, 128), _f32))(
            key_emb, str_e, nums2, Wb1, Wb2, wnum_row, cprime)


def _tc_layer(p0, p1, h, invd, W, b2):
    def body(p0_ref, p1_ref, h_ref, d_ref, w_ref, b_ref, out_ref):
        agg = (p0_ref[...] + p1_ref[...] + h_ref[...]) * d_ref[...]
        out_ref[...] = jnp.maximum(
            jnp.dot(agg, w_ref[...], preferred_element_type=_f32) + b_ref[...], 0.0)

    return pl.pallas_call(
        body, out_shape=jax.ShapeDtypeStruct((N, 128), _f32))(p0, p1, h, invd, W, b2)


def _tc_layer_head(p0, p1, h, invd, W, b2, Wh_, bh2):
    def body(p0_ref, p1_ref, h_ref, d_ref, w_ref, b_ref, wh_ref, bh_ref, out_ref):
        agg = (p0_ref[...] + p1_ref[...] + h_ref[...]) * d_ref[...]
        h2 = jnp.maximum(
            jnp.dot(agg, w_ref[...], preferred_element_type=_f32) + b_ref[...], 0.0)
        out_ref[...] = jnp.dot(h2, wh_ref[...], preferred_element_type=_f32) + bh_ref[...]

    return pl.pallas_call(
        body, out_shape=jax.ShapeDtypeStruct((N, 128), _f32))(
            p0, p1, h, invd, W, b2, Wh_, bh2)


def _pad_ids(ids, k):
    tot = NW * k * 128
    ids = ids.astype(_i32)
    return jnp.concatenate(
        [ids, jnp.zeros((tot - ids.shape[0],), _i32)]).reshape(NW, k, 128)


def kernel(edge_index, types_tokens, node_types, strings, numbers, readout_idx,
           str_table, Wx, Wh, bh, W_num, b_num, W_bn, b_bn, W1, b1, W2, b2,
           W_head, b_head):
    src = edge_index[0].astype(_i32)
    dst = edge_index[1].astype(_i32)
    npad = E_PAD - src.shape[0]
    src3 = jnp.concatenate([src, jnp.zeros((npad,), _i32)]).reshape(NW, 80, 128)
    dst_p = jnp.concatenate([dst, jnp.full((npad,), DUMMY, _i32)])
    dst3 = dst_p.reshape(NW, 80, 128)
    dst16 = dst_p.reshape(NS, 160, 128)

    tok_ids = jnp.swapaxes(types_tokens, 0, 1).reshape(-1)
    ids0 = jnp.concatenate([tok_ids.astype(_i32), strings.astype(_i32)])
    gath0 = _sc_gather(str_table, _pad_ids(ids0, 4), 4)
    tok = gath0[:4096]
    str_e = gath0[4096:4096 + N]

    invd = _sc_deg(dst16)[:N]

    key_tab = _tc_rnn(tok, Wx, Wh, bh.reshape(1, D))

    key_emb = _sc_gather(key_tab, _pad_ids(node_types, 3), 3)[:N]

    Wb1 = W_bn[:D]
    Wb2 = W_bn[D:2 * D]
    wnum_row = (W_num @ W_bn[2 * D:]).reshape(1, D)
    cprime = (b_bn + b_num @ W_bn[2 * D:]).reshape(1, D)
    bn = _tc_bn(key_emb, str_e, numbers.reshape(N, 1), Wb1, Wb2, wnum_row, cprime)

    parts1 = _sc_msg(bn, src3, dst3)
    h1 = _tc_layer(parts1[0, :N], parts1[1, :N], bn, invd, W1, b1.reshape(1, D))

    parts2 = _sc_msg(h1, src3, dst3)
    z = _tc_layer_head(parts2[0, :N], parts2[1, :N], h1, invd,
                       W2, b2.reshape(1, D), W_head, b_head.reshape(1, D))

    out = _sc_gather(z, _pad_ids(readout_idx, 1), 1)[:readout_idx.shape[0]]
    return out


# speedup vs baseline: 3.3115x; 1.0629x over previous
import functools

import jax
import jax.numpy as jnp
from jax import lax
from jax.experimental import pallas as pl
from jax.experimental.pallas import tpu as pltpu
from jax.experimental.pallas import tpu_sc as plsc

N = 10000
D = 128
NC = 2
NS = 16
NW = NC * NS
N_PAD = 10240
DUMMY = N
E_PAD = 327680
ROWS_PER_TILE = N_PAD // NS
CHUNKS_PER_TILE = ROWS_PER_TILE // 128

_f32 = jnp.float32
_i32 = jnp.int32


def _mesh():
    return plsc.VectorSubcoreMesh(core_axis_name="c", subcore_axis_name="s")


def _zero_rows(buf, nrows, ncol16):
    def row(r, _):
        for q in range(ncol16):
            buf[r, pl.ds(q * 16, 16)] = jnp.zeros((16,), _f32)
        return 0
    lax.fori_loop(0, nrows, row, 0)


def _sc_gather(table, idx3, k):
    nout = NW * k * 128

    @functools.partial(
        pl.kernel,
        out_type=jax.ShapeDtypeStruct((nout, 128), _f32),
        mesh=_mesh(),
        scratch_types=[
            pltpu.VMEM((k, 128), _i32),
            pltpu.VMEM((128, 128), _f32),
            pltpu.SemaphoreType.DMA,
        ],
    )
    def kfn(table_h, idx_h, out_h, idx_v, buf, sem):
        cid = lax.axis_index("c")
        sid = lax.axis_index("s")
        w = cid * NS + sid
        pltpu.sync_copy(idx_h.at[w], idx_v)
        for j in range(k):
            pltpu.async_copy(table_h.at[idx_v.at[j]], buf, sem).wait()
            pltpu.sync_copy(buf, out_h.at[pl.ds((w * k + j) * 128, 128)])

    return kfn(table, idx3)


def _sc_deg(dst3):
    @functools.partial(
        pl.kernel,
        out_type=jax.ShapeDtypeStruct((NC, N_PAD, 128), _f32),
        mesh=_mesh(),
        scratch_types=[
            pltpu.VMEM_SHARED((N_PAD, 128), _f32),
            pltpu.VMEM((80, 128), _i32),
            pltpu.VMEM((128, 128), _f32),
        ],
    )
    def kfn(dst_h, out_h, deg_sh, dst_v, buf):
        cid = lax.axis_index("c")
        sid = lax.axis_index("s")
        w = cid * NS + sid
        _zero_rows(buf, 128, 8)
        base = sid * ROWS_PER_TILE
        for c in range(CHUNKS_PER_TILE):
            pltpu.sync_copy(buf, deg_sh.at[pl.ds(base + c * 128, 128)])
        plsc.subcore_barrier()
        pltpu.sync_copy(dst_h.at[w], dst_v)

        def initrow(r, _):
            for q in range(8):
                buf[r, pl.ds(q * 16, 16)] = jnp.ones((16,), _f32)
            return 0
        lax.fori_loop(0, 128, initrow, 0)

        def body(j, _):
            pltpu.sync_copy(buf, deg_sh.at[dst_v.at[j]], add=True)
            return 0
        lax.fori_loop(0, 80, body, 0)
        plsc.subcore_barrier()
        for c in range(CHUNKS_PER_TILE):
            pltpu.sync_copy(deg_sh.at[pl.ds(base + c * 128, 128)],
                            out_h.at[cid, pl.ds(base + c * 128, 128)])

    return kfn(dst3)


def _sc_msg(h, src3, dst3):
    @functools.partial(
        pl.kernel,
        out_type=jax.ShapeDtypeStruct((NC, N_PAD, 128), _f32),
        mesh=_mesh(),
        scratch_types=[
            pltpu.VMEM_SHARED((N_PAD, 128), _f32),
            pltpu.VMEM((80, 128), _i32),
            pltpu.VMEM((80, 128), _i32),
            pltpu.VMEM((128, 128), _f32),
            pltpu.SemaphoreType.DMA,
        ],
    )
    def kfn(h_h, src_h, dst_h, out_h, acc_sh, src_v, dst_v, buf, sem):
        cid = lax.axis_index("c")
        sid = lax.axis_index("s")
        w = cid * NS + sid
        _zero_rows(buf, 128, 8)
        base = sid * ROWS_PER_TILE
        for c in range(CHUNKS_PER_TILE):
            pltpu.sync_copy(buf, acc_sh.at[pl.ds(base + c * 128, 128)])
        plsc.subcore_barrier()
        pltpu.sync_copy(src_h.at[w], src_v)
        pltpu.sync_copy(dst_h.at[w], dst_v)

        def body(j, _):
            pltpu.async_copy(h_h.at[src_v.at[j]], buf, sem).wait()
            pltpu.sync_copy(buf, acc_sh.at[dst_v.at[j]], add=True)
            return 0
        lax.fori_loop(0, 80, body, 0)
        plsc.subcore_barrier()
        for c in range(CHUNKS_PER_TILE):
            pltpu.sync_copy(acc_sh.at[pl.ds(base + c * 128, 128)],
                            out_h.at[cid, pl.ds(base + c * 128, 128)])

    return kfn(h, src3, dst3)


def _tc_rnn(tok, Wx, Wh, bh2):
    def body(tok_ref, wx_ref, wh_ref, b_ref, out_ref):
        wx = wx_ref[...]
        wh = wh_ref[...]
        b = b_ref[...]
        h = jnp.zeros((512, 128), _f32)
        for l in range(8):
            x = tok_ref[pl.ds(l * 512, 512), :]
            h = jnp.tanh(jnp.dot(x, wx, preferred_element_type=_f32)
                         + jnp.dot(h, wh, preferred_element_type=_f32) + b)
        out_ref[...] = h

    return pl.pallas_call(
        body, out_shape=jax.ShapeDtypeStruct((512, 128), _f32))(tok, Wx, Wh, bh2)


def _tc_bn(key_emb, str_e, nums2, Wb1, Wb2, wnum_row, cprime):
    def body(k_ref, s_ref, n_ref, w1_ref, w2_ref, wn_ref, c_ref, out_ref):
        acc = jnp.dot(k_ref[...], w1_ref[...], preferred_element_type=_f32)
        acc += jnp.dot(s_ref[...], w2_ref[...], preferred_element_type=_f32)
        acc += n_ref[...] * wn_ref[...]
        acc += c_ref[...]
        out_ref[...] = jnp.maximum(acc, 0.0)

    return pl.pallas_call(
        body, out_shape=jax.ShapeDtypeStruct((N, 128), _f32))(
            key_emb, str_e, nums2, Wb1, Wb2, wnum_row, cprime)


def _inv_deg(d0_ref, d1_ref):
    deg = d0_ref[...] + d1_ref[...]
    return 1.0 / (1.0 + deg[:, 0:1])


def _tc_layer(p0, p1, h, d0, d1, W, b2):
    def body(p0_ref, p1_ref, h_ref, d0_ref, d1_ref, w_ref, b_ref, out_ref):
        agg = (p0_ref[...] + p1_ref[...] + h_ref[...]) * _inv_deg(d0_ref, d1_ref)
        out_ref[...] = jnp.maximum(
            jnp.dot(agg, w_ref[...], preferred_element_type=_f32) + b_ref[...], 0.0)

    return pl.pallas_call(
        body, out_shape=jax.ShapeDtypeStruct((N, 128), _f32))(
            p0, p1, h, d0, d1, W, b2)


def _tc_layer_head(p0, p1, h, d0, d1, W, b2, Wh_, bh2):
    def body(p0_ref, p1_ref, h_ref, d0_ref, d1_ref, w_ref, b_ref,
             wh_ref, bh_ref, out_ref):
        agg = (p0_ref[...] + p1_ref[...] + h_ref[...]) * _inv_deg(d0_ref, d1_ref)
        h2 = jnp.maximum(
            jnp.dot(agg, w_ref[...], preferred_element_type=_f32) + b_ref[...], 0.0)
        out_ref[...] = jnp.dot(h2, wh_ref[...], preferred_element_type=_f32) + bh_ref[...]

    return pl.pallas_call(
        body, out_shape=jax.ShapeDtypeStruct((N, 128), _f32))(
            p0, p1, h, d0, d1, W, b2, Wh_, bh2)


def _pad_ids(ids, k):
    tot = NW * k * 128
    ids = ids.astype(_i32)
    return jnp.concatenate(
        [ids, jnp.zeros((tot - ids.shape[0],), _i32)]).reshape(NW, k, 128)


def kernel(edge_index, types_tokens, node_types, strings, numbers, readout_idx,
           str_table, Wx, Wh, bh, W_num, b_num, W_bn, b_bn, W1, b1, W2, b2,
           W_head, b_head):
    src = edge_index[0].astype(_i32)
    dst = edge_index[1].astype(_i32)
    npad = E_PAD - src.shape[0]
    src3 = jnp.concatenate([src, jnp.zeros((npad,), _i32)]).reshape(NW, 80, 128)
    dst_p = jnp.concatenate([dst, jnp.full((npad,), DUMMY, _i32)])
    dst3 = dst_p.reshape(NW, 80, 128)

    tok_ids = jnp.swapaxes(types_tokens, 0, 1).reshape(-1)
    ids0 = jnp.concatenate([tok_ids.astype(_i32), strings.astype(_i32)])
    gath0 = _sc_gather(str_table, _pad_ids(ids0, 4), 4)
    tok = gath0[:4096]
    str_e = gath0[4096:4096 + N]
    degp = _sc_deg(dst3)
    d0 = degp[0, :N]
    d1 = degp[1, :N]

    key_tab = _tc_rnn(tok, Wx, Wh, bh.reshape(1, D))

    key_emb = _sc_gather(key_tab, _pad_ids(node_types, 3), 3)[:N]

    Wb1 = W_bn[:D]
    Wb2 = W_bn[D:2 * D]
    wnum_row = (W_num @ W_bn[2 * D:]).reshape(1, D)
    cprime = (b_bn + b_num @ W_bn[2 * D:]).reshape(1, D)
    bn = _tc_bn(key_emb, str_e, numbers.reshape(N, 1), Wb1, Wb2, wnum_row, cprime)

    parts1 = _sc_msg(bn, src3, dst3)
    h1 = _tc_layer(parts1[0, :N], parts1[1, :N], bn, d0, d1, W1, b1.reshape(1, D))

    parts2 = _sc_msg(h1, src3, dst3)
    z = _tc_layer_head(parts2[0, :N], parts2[1, :N], h1, d0, d1,
                       W2, b2.reshape(1, D), W_head, b_head.reshape(1, D))

    out = _sc_gather(z, _pad_ids(readout_idx, 1), 1)[:readout_idx.shape[0]]
    return out


# speedup vs baseline: 3.5532x; 1.0730x over previous
import functools

import jax
import jax.numpy as jnp
from jax import lax
from jax.experimental import pallas as pl
from jax.experimental.pallas import tpu as pltpu
from jax.experimental.pallas import tpu_sc as plsc

N = 10000
D = 128
NC = 2
NS = 16
NW = NC * NS
N_PAD = 10240
DUMMY = N
E_PAD = 327680
ROWS_PER_TILE = N_PAD // NS
CHUNKS_PER_TILE = ROWS_PER_TILE // 128

_f32 = jnp.float32
_i32 = jnp.int32


def _mesh():
    return plsc.VectorSubcoreMesh(core_axis_name="c", subcore_axis_name="s")


def _zero_rows(buf, nrows, ncol16):
    def row(r, _):
        for q in range(ncol16):
            buf[r, pl.ds(q * 16, 16)] = jnp.zeros((16,), _f32)
        return 0
    lax.fori_loop(0, nrows, row, 0)


def _sc_gather(table, idx3, k):
    nout = NW * k * 128

    @functools.partial(
        pl.kernel,
        out_type=jax.ShapeDtypeStruct((nout, 128), _f32),
        mesh=_mesh(),
        scratch_types=[
            pltpu.VMEM((k, 128), _i32),
            pltpu.VMEM((128, 128), _f32),
            pltpu.SemaphoreType.DMA,
        ],
    )
    def kfn(table_h, idx_h, out_h, idx_v, buf, sem):
        cid = lax.axis_index("c")
        sid = lax.axis_index("s")
        w = cid * NS + sid
        pltpu.sync_copy(idx_h.at[w], idx_v)
        for j in range(k):
            pltpu.async_copy(table_h.at[idx_v.at[j]], buf, sem).wait()
            pltpu.sync_copy(buf, out_h.at[pl.ds((w * k + j) * 128, 128)])

    return kfn(table, idx3)


def _sc_deg(dst3):
    @functools.partial(
        pl.kernel,
        out_type=jax.ShapeDtypeStruct((NC, N_PAD, 128), _f32),
        mesh=_mesh(),
        scratch_types=[
            pltpu.VMEM_SHARED((N_PAD, 128), _f32),
            pltpu.VMEM((80, 128), _i32),
            pltpu.VMEM((128, 128), _f32),
        ],
    )
    def kfn(dst_h, out_h, deg_sh, dst_v, buf):
        cid = lax.axis_index("c")
        sid = lax.axis_index("s")
        w = cid * NS + sid
        _zero_rows(buf, 128, 8)
        base = sid * ROWS_PER_TILE
        for c in range(CHUNKS_PER_TILE):
            pltpu.sync_copy(buf, deg_sh.at[pl.ds(base + c * 128, 128)])
        plsc.subcore_barrier()
        pltpu.sync_copy(dst_h.at[w], dst_v)

        def initrow(r, _):
            for q in range(8):
                buf[r, pl.ds(q * 16, 16)] = jnp.ones((16,), _f32)
            return 0
        lax.fori_loop(0, 128, initrow, 0)

        def body(j, _):
            pltpu.sync_copy(buf, deg_sh.at[dst_v.at[j]], add=True)
            return 0
        lax.fori_loop(0, 80, body, 0)
        plsc.subcore_barrier()
        for c in range(CHUNKS_PER_TILE):
            pltpu.sync_copy(deg_sh.at[pl.ds(base + c * 128, 128)],
                            out_h.at[cid, pl.ds(base + c * 128, 128)])

    return kfn(dst3)


def _sc_msg(h, src3, dst3):
    @functools.partial(
        pl.kernel,
        out_type=jax.ShapeDtypeStruct((NC, N_PAD, 128), _f32),
        mesh=_mesh(),
        scratch_types=[
            pltpu.VMEM_SHARED((N_PAD, 128), _f32),
            pltpu.VMEM((16, 128), _i32),
            pltpu.VMEM((16, 128), _i32),
            pltpu.VMEM((16, 128), _i32),
            pltpu.VMEM((16, 128), _i32),
            pltpu.VMEM((128, 128), _f32),
            pltpu.VMEM((128, 128), _f32),
            pltpu.SemaphoreType.DMA,
            pltpu.SemaphoreType.DMA,
            pltpu.SemaphoreType.DMA,
        ],
    )
    def kfn(h_h, src_h, dst_h, out_h, acc_sh, sv0, dv0, sv1, dv1,
            b0, b1, s0, s1, si):
        cid = lax.axis_index("c")
        sid = lax.axis_index("s")
        w = cid * NS + sid
        _zero_rows(b0, 128, 8)
        base = sid * ROWS_PER_TILE
        for c in range(CHUNKS_PER_TILE):
            pltpu.sync_copy(b0, acc_sh.at[pl.ds(base + c * 128, 128)])
        plsc.subcore_barrier()
        svs = (sv0, sv1)
        dvs = (dv0, dv1)
        bufs = (b0, b1)
        sems = (s0, s1)
        pltpu.sync_copy(src_h.at[w, pl.ds(0, 16)], sv0)
        pltpu.sync_copy(dst_h.at[w, pl.ds(0, 16)], dv0)
        for s in range(5):
            svc, dvc = svs[s % 2], dvs[s % 2]
            if s + 1 < 5:
                svn, dvn = svs[(s + 1) % 2], dvs[(s + 1) % 2]
                pltpu.async_copy(src_h.at[w, pl.ds((s + 1) * 16, 16)], svn, si)
                pltpu.async_copy(dst_h.at[w, pl.ds((s + 1) * 16, 16)], dvn, si)
            pltpu.async_copy(h_h.at[svc.at[0]], b0, s0)
            pltpu.async_copy(h_h.at[svc.at[1]], b1, s1)

            def inner(t, _):
                for b in range(2):
                    j = t * 2 + b
                    pltpu.make_async_copy(h_h.at[svc.at[j]], bufs[b],
                                          sems[b]).wait()
                    pltpu.sync_copy(bufs[b], acc_sh.at[dvc.at[j]], add=True)

                    @pl.when(j + 2 < 16)
                    def _():
                        pltpu.async_copy(h_h.at[svc.at[j + 2]], bufs[b], sems[b])
                return 0
            lax.fori_loop(0, 8, inner, 0)
            if s + 1 < 5:
                pltpu.make_async_copy(src_h.at[w, pl.ds(0, 16)], svn, si).wait()
                pltpu.make_async_copy(dst_h.at[w, pl.ds(0, 16)], dvn, si).wait()
        plsc.subcore_barrier()
        for c in range(CHUNKS_PER_TILE):
            pltpu.sync_copy(acc_sh.at[pl.ds(base + c * 128, 128)],
                            out_h.at[cid, pl.ds(base + c * 128, 128)])

    return kfn(h, src3, dst3)


def _tc_rnn(tok, Wx, Wh, bh2):
    def body(tok_ref, wx_ref, wh_ref, b_ref, out_ref):
        wx = wx_ref[...]
        wh = wh_ref[...]
        b = b_ref[...]
        h = jnp.zeros((512, 128), _f32)
        for l in range(8):
            x = tok_ref[pl.ds(l * 512, 512), :]
            h = jnp.tanh(jnp.dot(x, wx, preferred_element_type=_f32)
                         + jnp.dot(h, wh, preferred_element_type=_f32) + b)
        out_ref[...] = h

    return pl.pallas_call(
        body, out_shape=jax.ShapeDtypeStruct((512, 128), _f32))(tok, Wx, Wh, bh2)


def _tc_bn(key_emb, str_e, nums2, Wb1, Wb2, wnum_row, cprime):
    def body(k_ref, s_ref, n_ref, w1_ref, w2_ref, wn_ref, c_ref, out_ref):
        acc = jnp.dot(k_ref[...], w1_ref[...], preferred_element_type=_f32)
        acc += jnp.dot(s_ref[...], w2_ref[...], preferred_element_type=_f32)
        acc += n_ref[...] * wn_ref[...]
        acc += c_ref[...]
        out_ref[...] = jnp.maximum(acc, 0.0)

    return pl.pallas_call(
        body, out_shape=jax.ShapeDtypeStruct((N, 128), _f32))(
            key_emb, str_e, nums2, Wb1, Wb2, wnum_row, cprime)


def _inv_deg(d0_ref, d1_ref):
    deg = d0_ref[...] + d1_ref[...]
    return 1.0 / (1.0 + deg[:, 0:1])


def _tc_layer(p0, p1, h, d0, d1, W, b2):
    def body(p0_ref, p1_ref, h_ref, d0_ref, d1_ref, w_ref, b_ref, out_ref):
        agg = (p0_ref[...] + p1_ref[...] + h_ref[...]) * _inv_deg(d0_ref, d1_ref)
        out_ref[...] = jnp.maximum(
            jnp.dot(agg, w_ref[...], preferred_element_type=_f32) + b_ref[...], 0.0)

    return pl.pallas_call(
        body, out_shape=jax.ShapeDtypeStruct((N, 128), _f32))(
            p0, p1, h, d0, d1, W, b2)


def _tc_layer_head(p0, p1, h, d0, d1, W, b2, Wh_, bh2):
    def body(p0_ref, p1_ref, h_ref, d0_ref, d1_ref, w_ref, b_ref,
             wh_ref, bh_ref, out_ref):
        agg = (p0_ref[...] + p1_ref[...] + h_ref[...]) * _inv_deg(d0_ref, d1_ref)
        h2 = jnp.maximum(
            jnp.dot(agg, w_ref[...], preferred_element_type=_f32) + b_ref[...], 0.0)
        out_ref[...] = jnp.dot(h2, wh_ref[...], preferred_element_type=_f32) + bh_ref[...]

    return pl.pallas_call(
        body, out_shape=jax.ShapeDtypeStruct((N, 128), _f32))(
            p0, p1, h, d0, d1, W, b2, Wh_, bh2)


def _pad_ids(ids, k):
    tot = NW * k * 128
    ids = ids.astype(_i32)
    return jnp.concatenate(
        [ids, jnp.zeros((tot - ids.shape[0],), _i32)]).reshape(NW, k, 128)


def kernel(edge_index, types_tokens, node_types, strings, numbers, readout_idx,
           str_table, Wx, Wh, bh, W_num, b_num, W_bn, b_bn, W1, b1, W2, b2,
           W_head, b_head):
    src = edge_index[0].astype(_i32)
    dst = edge_index[1].astype(_i32)
    npad = E_PAD - src.shape[0]
    src3 = jnp.concatenate([src, jnp.zeros((npad,), _i32)]).reshape(NW, 80, 128)
    dst_p = jnp.concatenate([dst, jnp.full((npad,), DUMMY, _i32)])
    dst3 = dst_p.reshape(NW, 80, 128)

    tok_ids = jnp.swapaxes(types_tokens, 0, 1).reshape(-1)
    ids0 = jnp.concatenate([tok_ids.astype(_i32), strings.astype(_i32)])
    gath0 = _sc_gather(str_table, _pad_ids(ids0, 4), 4)
    tok = gath0[:4096]
    str_e = gath0[4096:4096 + N]
    degp = _sc_deg(dst3)
    d0 = degp[0, :N]
    d1 = degp[1, :N]

    key_tab = _tc_rnn(tok, Wx, Wh, bh.reshape(1, D))

    key_emb = _sc_gather(key_tab, _pad_ids(node_types, 3), 3)[:N]

    Wb1 = W_bn[:D]
    Wb2 = W_bn[D:2 * D]
    wnum_row = (W_num @ W_bn[2 * D:]).reshape(1, D)
    cprime = (b_bn + b_num @ W_bn[2 * D:]).reshape(1, D)
    bn = _tc_bn(key_emb, str_e, numbers.reshape(N, 1), Wb1, Wb2, wnum_row, cprime)

    parts1 = _sc_msg(bn, src3, dst3)
    h1 = _tc_layer(parts1[0, :N], parts1[1, :N], bn, d0, d1, W1, b1.reshape(1, D))

    parts2 = _sc_msg(h1, src3, dst3)
    z = _tc_layer_head(parts2[0, :N], parts2[1, :N], h1, d0, d1,
                       W2, b2.reshape(1, D), W_head, b_head.reshape(1, D))

    out = _sc_gather(z, _pad_ids(readout_idx, 1), 1)[:readout_idx.shape[0]]
    return out
